# baseline (speedup 1.0000x reference)
import jax
import jax.numpy as jnp
from jax.experimental import pallas as pl

TEMP = 0.5


def kernel(x, Wr1, br1, Wr2, br2, Wt1, bt1, Wt2, bt2, Wo, bo):
    hp = jax.lax.Precision.DEFAULT
    h = jax.nn.gelu(jnp.dot(x, Wr1, precision=hp) + br1, approximate=False)
    route_logits = (jnp.dot(h, Wr2, precision=hp) + br2) / TEMP
    tile_idx = jnp.argmax(route_logits, axis=-1)
    th = jax.nn.gelu(jnp.einsum('bd,tdf->btf', x, Wt1, precision=hp) + bt1[None, :, :], approximate=False)
    tile_outputs = jnp.einsum('btf,tfd->btd', th, Wt2, precision=hp) + bt2[None, :, :]
    sel = jnp.take_along_axis(tile_outputs, tile_idx[:, None, None], axis=1)[:, 0, :]
    output = jnp.dot(sel, Wo, precision=hp) + bo
    return (output, tile_idx, route_logits)


# speedup vs baseline: 5.9321x; 5.9321x over previous
import functools

import jax
import jax.numpy as jnp
from jax.experimental import pallas as pl
from jax.experimental.pallas import tpu as pltpu
from jax.experimental.pallas import tpu_sc as plsc

D = 1024
F = 2048
T = 8
NTOK = 4096
TEMP = 0.5

TB = 512
NB = NTOK // TB
B = 256
PAD_N = NTOK + T * B
NBLK = PAD_N // B
SC_W = 32


def _bdot(a, b):
    return jnp.dot(a.astype(jnp.bfloat16), b, preferred_element_type=jnp.float32)


def _gelu(x):
    return 0.5 * x * (1.0 + jax.lax.erf(x * jnp.float32(0.7071067811865476)))


def _router_body(x_ref, wr1_ref, br1_ref, wr2_ref, br2_ref, lg_ref, ti_ref):
    h = _bdot(x_ref[...], wr1_ref[...]) + br1_ref[...]
    h = _gelu(h)
    lg = (_bdot(h, wr2_ref[...]) + br2_ref[...]) / TEMP
    lg_ref[...] = lg
    m = jnp.max(lg, axis=1, keepdims=True)
    col = jax.lax.broadcasted_iota(jnp.int32, (TB, T), 1)
    ti_ref[...] = jnp.min(jnp.where(lg == m, col, T), axis=1, keepdims=True)


def _router(x, Wr1b, br1, Wr2b, br2):
    return pl.pallas_call(
        _router_body,
        grid=(NB,),
        in_specs=[
            pl.BlockSpec((TB, D), lambda j: (j, 0)),
            pl.BlockSpec((D, D), lambda j: (0, 0)),
            pl.BlockSpec((1, D), lambda j: (0, 0)),
            pl.BlockSpec((D, T), lambda j: (0, 0)),
            pl.BlockSpec((1, T), lambda j: (0, 0)),
        ],
        out_specs=[
            pl.BlockSpec((TB, T), lambda j: (j, 0)),
            pl.BlockSpec((TB, 1), lambda j: (j, 0)),
        ],
        out_shape=[
            jax.ShapeDtypeStruct((NTOK, T), jnp.float32),
            jax.ShapeDtypeStruct((NTOK, 1), jnp.int32),
        ],
    )(x, Wr1b, br1, Wr2b, br2)


def _dispatch_body(ti_ref, pos_ref, bexp_ref, bval_ref):
    lane8 = jax.lax.broadcasted_iota(jnp.int32, (1, T), 1)
    oh_full = (ti_ref[...] == lane8).astype(jnp.float32)
    counts = jnp.sum(oh_full, axis=0, keepdims=True).astype(jnp.int32)
    padded = (counts + (B - 1)) & ~(B - 1)
    starts = jnp.zeros((1, T), jnp.int32)
    for k in range(1, T):
        starts = starts + jnp.roll(padded, k, axis=1) * (lane8 >= k)
    used = jnp.sum(padded, axis=1, keepdims=True)

    r = jax.lax.broadcasted_iota(jnp.int32, (TB, TB), 0)
    c = jax.lax.broadcasted_iota(jnp.int32, (TB, TB), 1)
    tril = (r > c).astype(jnp.float32)
    running = starts.astype(jnp.float32)
    for j in range(NB):
        ohc = oh_full[j * TB:(j + 1) * TB, :]
        rank = _bdot(tril, ohc.astype(jnp.bfloat16))
        posc = jnp.sum((rank + running) * ohc, axis=1, keepdims=True)
        pos_ref[j * TB:(j + 1) * TB, :] = posc.astype(jnp.int32)
        running = running + jnp.sum(ohc, axis=0, keepdims=True)

    ends = starts + padded
    brow = jax.lax.broadcasted_iota(jnp.int32, (NBLK, 1), 0) * B
    nb_before = jnp.sum((brow >= ends).astype(jnp.int32), axis=1, keepdims=True)
    bexp_ref[...] = jnp.minimum(nb_before, T - 1)
    bval_ref[...] = (brow < used).astype(jnp.int32)


def _dispatch(tidx2d):
    return pl.pallas_call(
        _dispatch_body,
        out_shape=[
            jax.ShapeDtypeStruct((NTOK, 1), jnp.int32),
            jax.ShapeDtypeStruct((NBLK, 1), jnp.int32),
            jax.ShapeDtypeStruct((NBLK, 1), jnp.int32),
        ],
    )(tidx2d)


N_SUB = 32
ROWS_PER_SUB = NTOK // N_SUB
CH = 64


def _sc_scatter(x, pos_row):
    mesh = plsc.VectorSubcoreMesh(core_axis_name="core", subcore_axis_name="subcore")

    @functools.partial(
        pl.kernel,
        out_type=jax.ShapeDtypeStruct((PAD_N, D), jnp.float32),
        mesh=mesh,
        scratch_types=[
            pltpu.VMEM((1, ROWS_PER_SUB), jnp.int32),
            pltpu.VMEM((CH, D), jnp.float32),
            pltpu.SemaphoreType.DMA,
        ],
    )
    def kernel(x_hbm, i_hbm, o_hbm, idx_buf, dbuf, sem):
        g = jax.lax.axis_index("core") * 16 + jax.lax.axis_index("subcore")
        row0 = g * ROWS_PER_SUB
        pltpu.async_copy(i_hbm.at[:, pl.ds(row0, ROWS_PER_SUB)], idx_buf, sem).wait()
        for c in range(ROWS_PER_SUB // CH):
            pltpu.async_copy(x_hbm.at[pl.ds(row0 + c * CH, CH), :], dbuf, sem).wait()
            pltpu.async_copy(dbuf, o_hbm.at[idx_buf.at[0, pl.ds(c * CH, CH)]], sem).wait()

    return kernel(x, pos_row)


def _sc_gather(y, pos_row):
    mesh = plsc.VectorSubcoreMesh(core_axis_name="core", subcore_axis_name="subcore")

    @functools.partial(
        pl.kernel,
        out_type=jax.ShapeDtypeStruct((NTOK, D), jnp.float32),
        mesh=mesh,
        scratch_types=[
            pltpu.VMEM((1, ROWS_PER_SUB), jnp.int32),
            pltpu.VMEM((CH, D), jnp.float32),
            pltpu.SemaphoreType.DMA,
        ],
    )
    def kernel(y_hbm, i_hbm, o_hbm, idx_buf, dbuf, sem):
        g = jax.lax.axis_index("core") * 16 + jax.lax.axis_index("subcore")
        row0 = g * ROWS_PER_SUB
        pltpu.async_copy(i_hbm.at[:, pl.ds(row0, ROWS_PER_SUB)], idx_buf, sem).wait()
        for c in range(ROWS_PER_SUB // CH):
            pltpu.async_copy(y_hbm.at[idx_buf.at[0, pl.ds(c * CH, CH)]], dbuf, sem).wait()
            pltpu.async_copy(dbuf, o_hbm.at[pl.ds(row0 + c * CH, CH), :], sem).wait()

    return kernel(y, pos_row)


def _ffn_body(bexp_ref, bval_ref, x_ref, w1_ref, b1_ref, w2_ref, b2_ref,
              wo_ref, bo_ref, y_ref):
    j = pl.program_id(0)

    @pl.when(bval_ref[j] == 1)
    def _():
        t = _bdot(x_ref[...], w1_ref[0]) + b1_ref[0]
        t = _gelu(t)
        s = _bdot(t, w2_ref[0]) + b2_ref[0]
        y_ref[...] = _bdot(s, wo_ref[...]) + bo_ref[...]


def _ffn(x_sorted, W1b, bt1, W2b, bt2, Wob, bo, bexp, bval):
    grid_spec = pltpu.PrefetchScalarGridSpec(
        num_scalar_prefetch=2,
        grid=(NBLK,),
        in_specs=[
            pl.BlockSpec((B, D), lambda j, be, bv: (j, 0)),
            pl.BlockSpec((1, D, F), lambda j, be, bv: (be[j], 0, 0)),
            pl.BlockSpec((1, 1, F), lambda j, be, bv: (be[j], 0, 0)),
            pl.BlockSpec((1, F, D), lambda j, be, bv: (be[j], 0, 0)),
            pl.BlockSpec((1, 1, D), lambda j, be, bv: (be[j], 0, 0)),
            pl.BlockSpec((D, D), lambda j, be, bv: (0, 0)),
            pl.BlockSpec((1, D), lambda j, be, bv: (0, 0)),
        ],
        out_specs=pl.BlockSpec((B, D), lambda j, be, bv: (j, 0)),
    )
    return pl.pallas_call(
        _ffn_body,
        grid_spec=grid_spec,
        out_shape=jax.ShapeDtypeStruct((PAD_N, D), jnp.float32),
    )(bexp, bval, x_sorted, W1b, bt1, W2b, bt2, Wob, bo)


def kernel(x, Wr1, br1, Wr2, br2, Wt1, bt1, Wt2, bt2, Wo, bo):
    bf = jnp.bfloat16
    logits, tidx2d = _router(x, Wr1.astype(bf), br1.reshape(1, D),
                             Wr2.astype(bf), br2.reshape(1, T))
    pos2d, bexp2, bval2 = _dispatch(tidx2d)
    pos_row = pos2d.reshape(1, NTOK)
    x_sorted = _sc_scatter(x, pos_row)
    y = _ffn(x_sorted, Wt1.astype(bf), bt1.reshape(T, 1, F),
             Wt2.astype(bf), bt2.reshape(T, 1, D),
             Wo.astype(bf), bo.reshape(1, D),
             bexp2.reshape(NBLK), bval2.reshape(NBLK))
    output = _sc_gather(y, pos_row)
    return (output, tidx2d.reshape(NTOK), logits)
